# baseline (speedup 1.0000x reference)
import functools

import jax
import jax.numpy as jnp
from jax import lax
from jax.experimental import pallas as pl
from jax.experimental.pallas import tpu as pltpu
from jax.experimental.pallas import tpu_sc as plsc

B = 16384
F_DENSE = 64
VOCAB_S = 100
VOCAB_T = 968
DIM = 32
NC = 2
NS = 16
NW = NC * NS
ROWS_PER_W = B // NW
GROUPS = ROWS_PER_W // 16


@functools.partial(
    pl.kernel,
    out_type=(jax.ShapeDtypeStruct((B, DIM), jnp.float32),
              jax.ShapeDtypeStruct((B, DIM), jnp.float32)),
    mesh=plsc.VectorSubcoreMesh(core_axis_name="c", subcore_axis_name="s"),
    compiler_params=pltpu.CompilerParams(use_tc_tiling_on_sc=False),
    scratch_types=[
        pltpu.VMEM((ROWS_PER_W,), jnp.float32),
        pltpu.VMEM((ROWS_PER_W,), jnp.float32),
        pltpu.VMEM((ROWS_PER_W,), jnp.int32),
        pltpu.VMEM((ROWS_PER_W,), jnp.int32),
        pltpu.VMEM((ROWS_PER_W, DIM), jnp.float32),
        pltpu.VMEM((ROWS_PER_W, DIM), jnp.float32),
        pltpu.VMEM_SHARED((VOCAB_S + VOCAB_T, DIM), jnp.float32),
        pltpu.SemaphoreType.DMA,
    ],
)
def _sc_embed(sid_hbm, tid_hbm, w_s_hbm, w_t_hbm, out_s_hbm, out_t_hbm,
              sid_v, tid_v, idx_s_v, idx_t_v, emb_s_v, emb_t_v, tbl_sh, sem):
    sub = lax.axis_index("s")
    wid = sub * NC + lax.axis_index("c")
    base = wid * ROWS_PER_W
    rows = pl.ds(base, ROWS_PER_W)

    ids_s = pltpu.async_copy(sid_hbm.at[rows], sid_v, sem)
    ids_t = pltpu.async_copy(tid_hbm.at[rows], tid_v, sem)

    @pl.when(sub == 0)
    def _stage_tables():
        pltpu.sync_copy(w_s_hbm, tbl_sh.at[pl.ds(0, VOCAB_S)])
        pltpu.sync_copy(w_t_hbm, tbl_sh.at[pl.ds(VOCAB_S, VOCAB_T)])

    ids_s.wait()
    ids_t.wait()

    def _convert(g, carry):
        sl = pl.ds(g * 16, 16)
        idx_s_v[sl] = sid_v[sl].astype(jnp.int32)
        idx_t_v[sl] = tid_v[sl].astype(jnp.int32) + VOCAB_S
        return carry

    lax.fori_loop(0, GROUPS, _convert, 0, unroll=4)

    plsc.subcore_barrier()

    g_s = pltpu.async_copy(tbl_sh.at[idx_s_v], emb_s_v, sem)
    g_t = pltpu.async_copy(tbl_sh.at[idx_t_v], emb_t_v, sem)
    g_s.wait()
    g_t.wait()

    out_s = pltpu.async_copy(emb_s_v, out_s_hbm.at[rows], sem)
    out_t = pltpu.async_copy(emb_t_v, out_t_hbm.at[rows], sem)
    out_s.wait()
    out_t.wait()


def kernel(x, W_s, W_t):
    emb_s, emb_t = _sc_embed(x[:, F_DENSE], x[:, F_DENSE + 1], W_s, W_t)
    return jnp.concatenate((x[:, :F_DENSE], emb_s, emb_t), axis=1)
